# baseline (speedup 1.0000x reference)
import jax
import jax.numpy as jnp
from jax.experimental import pallas as pl
from jax.experimental.pallas import tpu as pltpu

_BLOCK_ROWS = 3336


def _mlp_body(x_ref, w1_ref, b1_ref, w2_ref, b2_ref, o_ref):
    h = jnp.dot(x_ref[:], w1_ref[:],
                preferred_element_type=jnp.float32)
    h = h + b1_ref[:]
    h = jnp.where(h > 0, h, jnp.exp(h) - 1.0)
    o = jnp.dot(h, w2_ref[:],
                preferred_element_type=jnp.float32)
    o_ref[:] = o + b2_ref[:]


def kernel(x, edge_index, W1, b1, W2, b2):
    del edge_index
    N, D = x.shape
    b1_2d = b1.reshape(1, D)
    b2_2d = b2.reshape(1, D)
    grid = (pl.cdiv(N, _BLOCK_ROWS),)
    return pl.pallas_call(
        _mlp_body,
        grid=grid,
        in_specs=[
            pl.BlockSpec((_BLOCK_ROWS, D), lambda i: (i, 0)),
            pl.BlockSpec((D, D), lambda i: (0, 0)),
            pl.BlockSpec((1, D), lambda i: (0, 0)),
            pl.BlockSpec((D, D), lambda i: (0, 0)),
            pl.BlockSpec((1, D), lambda i: (0, 0)),
        ],
        out_specs=pl.BlockSpec((_BLOCK_ROWS, D), lambda i: (i, 0)),
        out_shape=jax.ShapeDtypeStruct((N, D), x.dtype),
        compiler_params=pltpu.CompilerParams(
            dimension_semantics=("arbitrary",)),
    )(x, W1, b1_2d, W2, b2_2d)


# speedup vs baseline: 1.1247x; 1.1247x over previous
import jax
import jax.numpy as jnp
from jax.experimental import pallas as pl
from jax.experimental.pallas import tpu as pltpu

_BLOCK_ROWS = 4000


def _mlp_body(x_ref, w1_ref, b1_ref, w2_ref, b2_ref, o_ref):
    h = jnp.dot(x_ref[:].astype(jnp.bfloat16), w1_ref[:].astype(jnp.bfloat16),
                preferred_element_type=jnp.float32)
    h = h + b1_ref[:]
    h = jnp.where(h > 0, h, jnp.exp(h) - 1.0)
    o = jnp.dot(h.astype(jnp.bfloat16), w2_ref[:].astype(jnp.bfloat16),
                preferred_element_type=jnp.float32)
    o_ref[:] = o + b2_ref[:]


def kernel(x, edge_index, W1, b1, W2, b2):
    del edge_index
    N, D = x.shape
    b1_2d = b1.reshape(1, D)
    b2_2d = b2.reshape(1, D)
    grid = (pl.cdiv(N, _BLOCK_ROWS),)
    return pl.pallas_call(
        _mlp_body,
        grid=grid,
        in_specs=[
            pl.BlockSpec((_BLOCK_ROWS, D), lambda i: (i, 0)),
            pl.BlockSpec((D, D), lambda i: (0, 0)),
            pl.BlockSpec((1, D), lambda i: (0, 0)),
            pl.BlockSpec((D, D), lambda i: (0, 0)),
            pl.BlockSpec((1, D), lambda i: (0, 0)),
        ],
        out_specs=pl.BlockSpec((_BLOCK_ROWS, D), lambda i: (i, 0)),
        out_shape=jax.ShapeDtypeStruct((N, D), x.dtype),
        compiler_params=pltpu.CompilerParams(
            dimension_semantics=("arbitrary",)),
    )(x, W1, b1_2d, W2, b2_2d)


# speedup vs baseline: 1.1851x; 1.0537x over previous
import jax
import jax.numpy as jnp
from jax.experimental import pallas as pl
from jax.experimental.pallas import tpu as pltpu

_BLOCK_ROWS = 4480


def _mlp_body(x_ref, w1_ref, b1_ref, w2_ref, b2_ref, o_ref):
    h = jnp.dot(x_ref[:], w1_ref[:],
                preferred_element_type=jnp.float32)
    h = h + b1_ref[:]
    h = jnp.where(h > 0, h, jnp.exp(h) - 1.0)
    o = jnp.dot(h, w2_ref[:],
                preferred_element_type=jnp.float32)
    o_ref[:] = o + b2_ref[:]


def kernel(x, edge_index, W1, b1, W2, b2):
    del edge_index
    N, D = x.shape
    b1_2d = b1.reshape(1, D)
    b2_2d = b2.reshape(1, D)
    grid = (pl.cdiv(N, _BLOCK_ROWS),)
    return pl.pallas_call(
        _mlp_body,
        grid=grid,
        in_specs=[
            pl.BlockSpec((_BLOCK_ROWS, D), lambda i: (i, 0)),
            pl.BlockSpec((D, D), lambda i: (0, 0)),
            pl.BlockSpec((1, D), lambda i: (0, 0)),
            pl.BlockSpec((D, D), lambda i: (0, 0)),
            pl.BlockSpec((1, D), lambda i: (0, 0)),
        ],
        out_specs=pl.BlockSpec((_BLOCK_ROWS, D), lambda i: (i, 0)),
        out_shape=jax.ShapeDtypeStruct((N, D), x.dtype),
        compiler_params=pltpu.CompilerParams(
            dimension_semantics=("arbitrary",)),
    )(x, W1, b1_2d, W2, b2_2d)


# speedup vs baseline: 1.2150x; 1.0253x over previous
import jax
import jax.numpy as jnp
from jax.experimental import pallas as pl
from jax.experimental.pallas import tpu as pltpu

_BLOCK_ROWS = 4800


def _mlp_body(x_ref, w1_ref, b1_ref, w2_ref, b2_ref, o_ref):
    h = jnp.dot(x_ref[:], w1_ref[:],
                preferred_element_type=jnp.float32)
    h = h + b1_ref[:]
    h = jnp.where(h > 0, h, jnp.exp(h) - 1.0)
    o = jnp.dot(h, w2_ref[:],
                preferred_element_type=jnp.float32)
    o_ref[:] = o + b2_ref[:]


def kernel(x, edge_index, W1, b1, W2, b2):
    del edge_index
    N, D = x.shape
    b1_2d = b1.reshape(1, D)
    b2_2d = b2.reshape(1, D)
    grid = (pl.cdiv(N, _BLOCK_ROWS),)
    return pl.pallas_call(
        _mlp_body,
        grid=grid,
        in_specs=[
            pl.BlockSpec((_BLOCK_ROWS, D), lambda i: (i, 0)),
            pl.BlockSpec((D, D), lambda i: (0, 0)),
            pl.BlockSpec((1, D), lambda i: (0, 0)),
            pl.BlockSpec((D, D), lambda i: (0, 0)),
            pl.BlockSpec((1, D), lambda i: (0, 0)),
        ],
        out_specs=pl.BlockSpec((_BLOCK_ROWS, D), lambda i: (i, 0)),
        out_shape=jax.ShapeDtypeStruct((N, D), x.dtype),
        compiler_params=pltpu.CompilerParams(
            dimension_semantics=("arbitrary",)),
    )(x, W1, b1_2d, W2, b2_2d)
